# baseline (speedup 1.0000x reference)
import functools

import jax
import jax.numpy as jnp
from jax import lax
from jax.experimental import pallas as pl
from jax.experimental.pallas import tpu as pltpu
from jax.experimental.pallas import tpu_sc as plsc

D = 64
L = 16
NW = 32
SEQ = 200
BATCH = 1024
SEQ_PER_W = BATCH // NW
G_FULL = SEQ // L
TAIL = SEQ - L
SPLIT = 104


def _sc_body(xi_hbm, tt_hbm, table_hbm, seg_hbm, pe_hbm, out_hbm,
             xi_v, tt_v, seg_v, add_v, pidx_v, pair_v, out_v, sem_rows):
    wid = lax.axis_index("s") * 2 + lax.axis_index("c")
    seq0 = wid * SEQ_PER_W

    pltpu.sync_copy(xi_hbm.at[pl.ds(seq0, SEQ_PER_W)], xi_v)
    pltpu.sync_copy(tt_hbm.at[pl.ds(seq0, SEQ_PER_W)], tt_v)
    pltpu.sync_copy(pe_hbm.at[pl.ds(0, SEQ)], out_v)
    pltpu.sync_copy(seg_hbm, seg_v)

    seg0_ = [seg_v[0, pl.ds(d * L, L)] for d in range(4)]
    seg1_ = [seg_v[1, pl.ds(d * L, L)] for d in range(4)]

    def build_add(s, _):
        for d in range(4):
            p = out_v[s, pl.ds(d * L, L)]
            add_v[s, pl.ds(d * L, L)] = p + seg0_[d]
            add_v[s, pl.ds(D + d * L, L)] = p + seg1_[d]
        return 0

    lax.fori_loop(0, SEQ, build_add, 0)

    group_offs = [g * L for g in range(G_FULL)] + [TAIL]

    def seq_body(q, _):
        for off in group_offs:
            pidx_v[pl.ds(off, L)] = lax.shift_right_logical(
                xi_v[q, pl.ds(off, L)], 1)

        cp0 = pltpu.async_copy(
            table_hbm.at[pidx_v.at[pl.ds(0, SPLIT)]],
            pair_v.at[pl.ds(0, SPLIT)], sem_rows)
        cp1 = pltpu.async_copy(
            table_hbm.at[pidx_v.at[pl.ds(SPLIT, SEQ - SPLIT)]],
            pair_v.at[pl.ds(SPLIT, SEQ - SPLIT)], sem_rows)
        cp0.wait()
        cp1.wait()

        for off in group_offs:
            tvec = tt_v[q, pl.ds(off, L)]
            hvec = xi_v[q, pl.ds(off, L)]
            for r16 in range(L):
                r = off + r16
                toff = tvec[r16] * D
                half = (hvec[r16] & 1) * D
                for d in range(4):
                    out_v[r, pl.ds(d * L, L)] = (
                        pair_v[r, pl.ds(half + d * L, L)]
                        + add_v[r, pl.ds(toff + d * L, L)])

        pltpu.sync_copy(out_v, out_hbm.at[seq0 + q])
        return 0

    lax.fori_loop(0, SEQ_PER_W, seq_body, 0)


@jax.jit
def _encode(xi, tt, table2, segment_table, pe2d):
    mesh = plsc.VectorSubcoreMesh(
        core_axis_name="c", subcore_axis_name="s", num_cores=2, num_subcores=16)
    run = pl.kernel(
        _sc_body,
        out_type=jax.ShapeDtypeStruct((BATCH, SEQ, D), jnp.float32),
        mesh=mesh,
        scratch_types=[
            pltpu.VMEM((SEQ_PER_W, SEQ), jnp.int32),
            pltpu.VMEM((SEQ_PER_W, SEQ), jnp.int32),
            pltpu.VMEM((2, D), jnp.float32),
            pltpu.VMEM((SEQ, 2 * D), jnp.float32),
            pltpu.VMEM((SEQ,), jnp.int32),
            pltpu.VMEM((SEQ, 2 * D), jnp.float32),
            pltpu.VMEM((SEQ, D), jnp.float32),
            pltpu.SemaphoreType.DMA,
        ],
    )
    return run(xi, tt, table2, segment_table, pe2d)


def kernel(x, token_types, token_table, segment_table, pe):
    xi = x.astype(jnp.int32)
    tt = token_types.astype(jnp.int32)
    table2 = token_table.reshape(token_table.shape[0] // 2, 2 * D)
    pe2d = pe.reshape(pe.shape[-2], D)
    return _encode(xi, tt, table2, segment_table, pe2d)


# speedup vs baseline: 1.0447x; 1.0447x over previous
import functools

import jax
import jax.numpy as jnp
from jax import lax
from jax.experimental import pallas as pl
from jax.experimental.pallas import tpu as pltpu
from jax.experimental.pallas import tpu_sc as plsc

D = 64
L = 16
NW = 32
SEQ = 200
BATCH = 1024
B_PER_W = BATCH // NW
G_FULL = SEQ // L
TAIL = SEQ - L
SPLIT = 104


def _sc_body(xi_hbm, tt_hbm, table_hbm, seg_hbm, pe_hbm, out_hbm,
             xi_v, tt_v, seg_v, add_v, rows_v, out_v, sem_rows):
    wid = lax.axis_index("s") * 2 + lax.axis_index("c")
    b0 = wid * B_PER_W

    pltpu.sync_copy(xi_hbm.at[pl.ds(b0, B_PER_W)], xi_v)
    pltpu.sync_copy(tt_hbm.at[pl.ds(b0, B_PER_W)], tt_v)
    pltpu.sync_copy(pe_hbm.at[pl.ds(0, SEQ)], out_v)
    pltpu.sync_copy(seg_hbm, seg_v)

    seg0_ = [seg_v[0, pl.ds(d * L, L)] for d in range(4)]
    seg1_ = [seg_v[1, pl.ds(d * L, L)] for d in range(4)]

    def build_add(s, _):
        for d in range(4):
            p = out_v[s, pl.ds(d * L, L)]
            add_v[s, pl.ds(d * L, L)] = p + seg0_[d]
            add_v[s, pl.ds(D + d * L, L)] = p + seg1_[d]
        return 0

    lax.fori_loop(0, SEQ, build_add, 0)

    group_offs = [g * L for g in range(G_FULL)] + [TAIL]

    def row_body(q, _):
        cp0 = pltpu.async_copy(
            table_hbm.at[xi_v.at[q, pl.ds(0, SPLIT)]],
            rows_v.at[pl.ds(0, SPLIT)], sem_rows)
        cp1 = pltpu.async_copy(
            table_hbm.at[xi_v.at[q, pl.ds(SPLIT, SEQ - SPLIT)]],
            rows_v.at[pl.ds(SPLIT, SEQ - SPLIT)], sem_rows)
        cp0.wait()
        cp1.wait()

        for off in group_offs:
            tvec = tt_v[q, pl.ds(off, L)]
            for r16 in range(L):
                r = off + r16
                toff = tvec[r16] * D
                for d in range(4):
                    out_v[r, pl.ds(d * L, L)] = (
                        rows_v[r, pl.ds(d * L, L)]
                        + add_v[r, pl.ds(toff + d * L, L)])

        pltpu.sync_copy(out_v, out_hbm.at[b0 + q])
        return 0

    lax.fori_loop(0, B_PER_W, row_body, 0)


@jax.jit
def _encode(xi, tt, table, segment_table, pe2d):
    mesh = plsc.VectorSubcoreMesh(
        core_axis_name="c", subcore_axis_name="s", num_cores=2, num_subcores=16)
    run = pl.kernel(
        _sc_body,
        out_type=jax.ShapeDtypeStruct((BATCH, SEQ, D), jnp.float32),
        mesh=mesh,
        compiler_params=pltpu.CompilerParams(use_tc_tiling_on_sc=False),
        scratch_types=[
            pltpu.VMEM((B_PER_W, SEQ), jnp.int32),
            pltpu.VMEM((B_PER_W, SEQ), jnp.int32),
            pltpu.VMEM((2, D), jnp.float32),
            pltpu.VMEM((SEQ, 2 * D), jnp.float32),
            pltpu.VMEM((SEQ, D), jnp.float32),
            pltpu.VMEM((SEQ, D), jnp.float32),
            pltpu.SemaphoreType.DMA,
        ],
    )
    return run(xi, tt, table, segment_table, pe2d)


def kernel(x, token_types, token_table, segment_table, pe):
    xi = x.astype(jnp.int32)
    tt = token_types.astype(jnp.int32)
    pe2d = pe.reshape(pe.shape[-2], D)
    return _encode(xi, tt, token_table, segment_table, pe2d)
